# Initial kernel scaffold; baseline (speedup 1.0000x reference)
import jax
import jax.numpy as jnp
from jax.experimental import pallas as pl


def kernel(edge_index_rates_book, edge_index_rates_movie, user_emb, book_emb, movie_emb, l1_ub_Ws, l1_ub_Wd, l1_ub_as, l1_ub_ad, l1_ub_b, l1_bu_Ws, l1_bu_Wd, l1_bu_as, l1_bu_ad, l1_bu_b, l1_um_Ws, l1_um_Wd, l1_um_as, l1_um_ad, l1_um_b, l1_mu_Ws, l1_mu_Wd, l1_mu_as, l1_mu_ad, l1_mu_b, l2_ub_Ws, l2_ub_Wd, l2_ub_as, l2_ub_ad, l2_ub_b, l2_bu_Ws, l2_bu_Wd, l2_bu_as, l2_bu_ad, l2_bu_b, l2_um_Ws, l2_um_Wd, l2_um_as, l2_um_ad, l2_um_b, l2_mu_Ws, l2_mu_Wd, l2_mu_as, l2_mu_ad, l2_mu_b, lin_u_W, lin_u_b, lin_b_W, lin_b_b, lin_m_W, lin_m_b):
    raise NotImplementedError("write your pallas kernel here")


# speedup vs baseline: 27.5734x; 27.5734x over previous
import functools
import jax
import jax.numpy as jnp
from jax import lax
from jax.experimental import pallas as pl
from jax.experimental.pallas import tpu as pltpu
from jax.experimental.pallas import tpu_sc as plsc

H = 2
HID = 32
NU = 100000
NB = 50000
NM = 50000
E = 800000

NCORE, NSUB, L = 2, 16, 16

EC = 80
SC_CH = 1000
WR_CH = 250
EPT = E // NSUB
NCHUNK = EPT // EC
EPS = 1e-16


def _ceil_div(a, b):
    return (a + b - 1) // b


@functools.lru_cache(maxsize=None)
def _conv_kernel(nsrc, ndst):
    half = ndst // 2

    def body(hs_hbm, ss_hbm, sd_hbm, src_hbm, dst_hbm, num_out, den_out,
             sidx, didx, lidx, ssv, sdv, exv, rows, msg, stg,
             ss_sh, sd_sh, num_sh, den_sh, sem, sem2):
        core = lax.axis_index("c")
        sub = lax.axis_index("s")

        nss = nsrc // SC_CH
        for k in range(_ceil_div(nss, NSUB)):
            c = sub + NSUB * k

            @pl.when(c < nss)
            def _():
                off = c * SC_CH
                pltpu.sync_copy(ss_hbm.at[pl.ds(off, SC_CH)],
                                stg.at[pl.ds(0, SC_CH)])
                pltpu.sync_copy(stg.at[pl.ds(0, SC_CH)],
                                ss_sh.at[pl.ds(off, SC_CH)])

        nsd = ndst // SC_CH
        for k in range(_ceil_div(nsd, NSUB)):
            c = sub + NSUB * k

            @pl.when(c < nsd)
            def _():
                off = c * SC_CH
                pltpu.sync_copy(sd_hbm.at[pl.ds(off, SC_CH)],
                                stg.at[pl.ds(0, SC_CH)])
                pltpu.sync_copy(stg.at[pl.ds(0, SC_CH)],
                                sd_sh.at[pl.ds(off, SC_CH)])

        zv = jnp.zeros((L,), jnp.float32)

        def zs(i, _):
            stg[pl.ds(i * L, L)] = zv
            return 0
        lax.fori_loop(0, stg.shape[0] // L, zs, 0)

        def zm(i, _):
            msg[i, pl.ds(0, L)] = zv
            msg[i, pl.ds(L, L)] = zv
            return 0
        lax.fori_loop(0, WR_CH, zm, 0)

        ndc = half // SC_CH
        for k in range(_ceil_div(ndc, NSUB)):
            c = sub + NSUB * k

            @pl.when(c < ndc)
            def _():
                pltpu.sync_copy(stg.at[pl.ds(0, SC_CH)],
                                den_sh.at[pl.ds(c * SC_CH, SC_CH)])

        nwc = half // WR_CH
        for k in range(_ceil_div(nwc, NSUB)):
            c = sub + NSUB * k

            @pl.when(c < nwc)
            def _():
                pltpu.sync_copy(msg, num_sh.at[pl.ds(c * WR_CH, WR_CH)])

        plsc.subcore_barrier()

        base = core * half

        def chunk(k, _):
            off = sub * EPT + k * EC
            pltpu.sync_copy(src_hbm.at[pl.ds(off, EC)], sidx)
            pltpu.sync_copy(dst_hbm.at[pl.ds(off, EC)], didx)
            c1 = pltpu.async_copy(ss_sh.at[sidx], ssv, sem)
            c2 = pltpu.async_copy(sd_sh.at[didx], sdv, sem)
            c3 = pltpu.async_copy(hs_hbm.at[sidx], rows, sem2)
            c1.wait()
            c2.wait()
            c3.wait()

            def grp(g, _):
                sv = ssv[pl.ds(g * L, L)]
                dv = sdv[pl.ds(g * L, L)]
                s = sv + dv
                e = jnp.where(s >= 0, s, 0.2 * s)
                ex = jnp.exp(e)
                exv[pl.ds(g * L, L)] = ex
                dl = didx[pl.ds(g * L, L)] - base
                okm = (dl >= 0) & (dl < half)
                lidx[0, pl.ds(g * L, L)] = jnp.where(okm, dl, half)
                for j in range(L):
                    sj = ex[j]
                    ei = g * L + j
                    msg[ei, pl.ds(0, L)] = sj * rows[ei, pl.ds(0, L)]
                    msg[ei, pl.ds(L, L)] = sj * rows[ei, pl.ds(L, L)]
                return 0

            lax.fori_loop(0, EC // L, grp, 0)
            pltpu.sync_copy(msg.at[pl.ds(0, EC)], num_sh.at[lidx.at[0]],
                            add=True)
            pltpu.sync_copy(exv, den_sh.at[lidx.at[0]], add=True)
            return 0

        lax.fori_loop(0, NCHUNK, chunk, 0)
        plsc.subcore_barrier()

        for k in range(_ceil_div(nwc, NSUB)):
            c = sub + NSUB * k

            @pl.when(c < nwc)
            def _():
                r0 = c * WR_CH
                pltpu.sync_copy(num_sh.at[pl.ds(r0, WR_CH)], msg)
                pltpu.sync_copy(msg, num_out.at[pl.ds(base + r0, WR_CH)])

        for k in range(_ceil_div(ndc, NSUB)):
            c = sub + NSUB * k

            @pl.when(c < ndc)
            def _():
                r0 = c * SC_CH
                pltpu.sync_copy(den_sh.at[pl.ds(r0, SC_CH)],
                                stg.at[pl.ds(0, SC_CH)])
                pltpu.sync_copy(stg.at[pl.ds(0, SC_CH)],
                                den_out.at[pl.ds(base + r0, SC_CH)])

    mesh = plsc.VectorSubcoreMesh(core_axis_name="c", subcore_axis_name="s",
                                  num_cores=NCORE, num_subcores=NSUB)
    return pl.kernel(
        body,
        out_type=[jax.ShapeDtypeStruct((ndst, HID), jnp.float32),
                  jax.ShapeDtypeStruct((ndst,), jnp.float32)],
        mesh=mesh,
        compiler_params=pltpu.CompilerParams(use_tc_tiling_on_sc=False),
        scratch_types=[
            pltpu.VMEM((EC,), jnp.int32),
            pltpu.VMEM((EC,), jnp.int32),
            pltpu.VMEM((1, EC), jnp.int32),
            pltpu.VMEM((EC,), jnp.float32),
            pltpu.VMEM((EC,), jnp.float32),
            pltpu.VMEM((EC,), jnp.float32),
            pltpu.VMEM((EC, HID), jnp.float32),
            pltpu.VMEM((WR_CH, HID), jnp.float32),
            pltpu.VMEM((1008,), jnp.float32),
            pltpu.MemorySpace.VMEM_SHARED((nsrc,), jnp.float32),
            pltpu.MemorySpace.VMEM_SHARED((ndst,), jnp.float32),
            pltpu.MemorySpace.VMEM_SHARED((ndst // 2 + 8, HID), jnp.float32),
            pltpu.MemorySpace.VMEM_SHARED((ndst // 2 + 8,), jnp.float32),
            pltpu.SemaphoreType.DMA,
            pltpu.SemaphoreType.DMA,
        ],
    )


_B = 1000


@functools.lru_cache(maxsize=None)
def _prep_kernel(n, parts):
    np_ = len(parts)

    def body(x_ref, w_ref, a_ref, *outs):
        x = x_ref[...]
        y = jnp.dot(x, w_ref[...], preferred_element_type=jnp.float32)
        oi = 0
        for i, kind in enumerate(parts):
            h0 = y[:, i * 2 * HID:i * 2 * HID + HID]
            h1 = y[:, i * 2 * HID + HID:(i + 1) * 2 * HID]
            s0 = jnp.sum(h0 * a_ref[2 * i][None, :], axis=1)
            s1 = jnp.sum(h1 * a_ref[2 * i + 1][None, :], axis=1)
            if kind == "p":
                outs[oi][...] = h0
                outs[oi + 1][...] = h1
                oi += 2
            outs[oi][...] = s0[:, None]
            outs[oi + 1][...] = s1[:, None]
            oi += 2

    out_shape = []
    out_specs = []
    for kind in parts:
        if kind == "p":
            for _ in range(H):
                out_shape.append(jax.ShapeDtypeStruct((n, HID), jnp.float32))
                out_specs.append(pl.BlockSpec((_B, HID), lambda i: (i, 0)))
        for _ in range(H):
            out_shape.append(jax.ShapeDtypeStruct((n, 1), jnp.float32))
            out_specs.append(pl.BlockSpec((_B, 1), lambda i: (i, 0)))

    return pl.pallas_call(
        body,
        grid=(n // _B,),
        in_specs=[
            pl.BlockSpec((_B, HID), lambda i: (i, 0)),
            pl.BlockSpec((HID, 2 * HID * np_), lambda i: (0, 0)),
            pl.BlockSpec((2 * np_, HID), lambda i: (0, 0)),
        ],
        out_specs=out_specs,
        out_shape=out_shape,
    )


@functools.lru_cache(maxsize=None)
def _combine_kernel(n, nconv, lin):
    def body(*refs):
        i = 0
        nums = []
        dens = []
        for _ in range(nconv * H):
            nums.append(refs[i])
            i += 1
        for _ in range(nconv * H):
            dens.append(refs[i])
            i += 1
        bsum = refs[i][...]
        i += 1
        if lin:
            lw = refs[i][...]
            lb = refs[i + 1][...]
            i += 2
        out = refs[i]
        pre = jnp.broadcast_to(bsum, (_B, HID)).astype(jnp.float32)
        for c in range(nconv):
            m0 = nums[2 * c][...] / (dens[2 * c][...] + EPS)
            m1 = nums[2 * c + 1][...] / (dens[2 * c + 1][...] + EPS)
            pre = pre + 0.5 * (m0 + m1)
        if lin:
            pre = jnp.dot(pre, lw, preferred_element_type=jnp.float32) + lb
        out[...] = jnp.maximum(pre, 0.0)

    in_specs = (
        [pl.BlockSpec((_B, HID), lambda i: (i, 0))] * (nconv * H)
        + [pl.BlockSpec((_B, 1), lambda i: (i, 0))] * (nconv * H)
        + [pl.BlockSpec((1, HID), lambda i: (0, 0))]
    )
    if lin:
        in_specs += [pl.BlockSpec((HID, HID), lambda i: (0, 0)),
                     pl.BlockSpec((1, HID), lambda i: (0, 0))]

    return pl.pallas_call(
        body,
        grid=(n // _B,),
        in_specs=in_specs,
        out_specs=pl.BlockSpec((_B, HID), lambda i: (i, 0)),
        out_shape=jax.ShapeDtypeStruct((n, HID), jnp.float32),
    )


def _gat_pair(hs, ss, sd, src, dst, nsrc, ndst):
    f = _conv_kernel(nsrc, ndst)
    nums = []
    dens = []
    for h in range(H):
        num, den = f(hs[h], ss[h].reshape(nsrc), sd[h].reshape(ndst),
                     src, dst)
        nums.append(num)
        dens.append(den.reshape(ndst, 1))
    return nums, dens


def _layer(xu, xb, xm, w, pfx, ei_ub, ei_um, lin=None):
    def cat(names_w, names_a):
        return (jnp.concatenate([w[pfx + n + "_W" + k] for n, k in names_w],
                                axis=1),
                jnp.concatenate([w[pfx + n + "_a" + k] for n, k in names_a],
                                axis=0))

    wu, au = cat([("ub", "s"), ("um", "s"), ("bu", "d"), ("mu", "d")],
                 [("ub", "s"), ("um", "s"), ("bu", "d"), ("mu", "d")])
    ou_ = _prep_kernel(NU, "ppss")(xu, wu, au)
    hs_ub, ss_ub = ou_[0:2], ou_[2:4]
    hs_um, ss_um = ou_[4:6], ou_[6:8]
    sd_bu, sd_mu = ou_[8:10], ou_[10:12]
    wb, ab = cat([("bu", "s"), ("ub", "d")], [("bu", "s"), ("ub", "d")])
    ob_ = _prep_kernel(NB, "ps")(xb, wb, ab)
    hs_bu, ss_bu, sd_ub = ob_[0:2], ob_[2:4], ob_[4:6]
    wm, am = cat([("mu", "s"), ("um", "d")], [("mu", "s"), ("um", "d")])
    om_ = _prep_kernel(NM, "ps")(xm, wm, am)
    hs_mu, ss_mu, sd_um = om_[0:2], om_[2:4], om_[4:6]

    nb_n, nb_d = _gat_pair(hs_ub, ss_ub, sd_ub, ei_ub[0], ei_ub[1], NU, NB)
    nm_n, nm_d = _gat_pair(hs_um, ss_um, sd_um, ei_um[0], ei_um[1], NU, NM)
    ub_n, ub_d = _gat_pair(hs_bu, ss_bu, sd_bu, ei_ub[1], ei_ub[0], NB, NU)
    um_n, um_d = _gat_pair(hs_mu, ss_mu, sd_mu, ei_um[1], ei_um[0], NM, NU)

    b_b = w[pfx + "ub_b"].reshape(1, HID)
    b_m = w[pfx + "um_b"].reshape(1, HID)
    b_u = (w[pfx + "bu_b"] + w[pfx + "mu_b"]).reshape(1, HID)
    if lin is None:
        cb = _combine_kernel(NB, 1, False)
        cu = _combine_kernel(NU, 2, False)
        ob = cb(*nb_n, *nb_d, b_b)
        om = cb(*nm_n, *nm_d, b_m)
        ou = cu(*ub_n, *um_n, *ub_d, *um_d, b_u)
    else:
        cb = _combine_kernel(NB, 1, True)
        cu = _combine_kernel(NU, 2, True)
        ob = cb(*nb_n, *nb_d, b_b, lin["b_W"], lin["b_b"].reshape(1, HID))
        om = cb(*nm_n, *nm_d, b_m, lin["m_W"], lin["m_b"].reshape(1, HID))
        ou = cu(*ub_n, *um_n, *ub_d, *um_d, b_u,
                lin["u_W"], lin["u_b"].reshape(1, HID))
    return ou, ob, om


def kernel(edge_index_rates_book, edge_index_rates_movie, user_emb, book_emb,
           movie_emb,
           l1_ub_Ws, l1_ub_Wd, l1_ub_as, l1_ub_ad, l1_ub_b,
           l1_bu_Ws, l1_bu_Wd, l1_bu_as, l1_bu_ad, l1_bu_b,
           l1_um_Ws, l1_um_Wd, l1_um_as, l1_um_ad, l1_um_b,
           l1_mu_Ws, l1_mu_Wd, l1_mu_as, l1_mu_ad, l1_mu_b,
           l2_ub_Ws, l2_ub_Wd, l2_ub_as, l2_ub_ad, l2_ub_b,
           l2_bu_Ws, l2_bu_Wd, l2_bu_as, l2_bu_ad, l2_bu_b,
           l2_um_Ws, l2_um_Wd, l2_um_as, l2_um_ad, l2_um_b,
           l2_mu_Ws, l2_mu_Wd, l2_mu_as, l2_mu_ad, l2_mu_b,
           lin_u_W, lin_u_b, lin_b_W, lin_b_b, lin_m_W, lin_m_b):
    w = dict(locals())
    wmap = {}
    for key, val in w.items():
        wmap[key] = val
    conv_w = {k: v for k, v in wmap.items()}

    ei_ub = edge_index_rates_book
    ei_um = edge_index_rates_movie

    u1, b1, m1 = _layer(user_emb, book_emb, movie_emb, conv_w, "l1_",
                        ei_ub, ei_um)
    lin = {"u_W": lin_u_W, "u_b": lin_u_b, "b_W": lin_b_W, "b_b": lin_b_b,
           "m_W": lin_m_W, "m_b": lin_m_b}
    ou, ob, om = _layer(u1, b1, m1, conv_w, "l2_", ei_ub, ei_um, lin=lin)
    return jnp.concatenate([ou, ob, om], axis=0)


# speedup vs baseline: 36.2528x; 1.3148x over previous
import functools
import jax
import jax.numpy as jnp
from jax import lax
from jax.experimental import pallas as pl
from jax.experimental.pallas import tpu as pltpu
from jax.experimental.pallas import tpu_sc as plsc

H = 2
HID = 32
NU = 100000
NB = 50000
NM = 50000
E = 800000

NCORE, NSUB, L = 2, 16, 16

EC = 80
SC_CH = 1000
WR_CH = 250
EPT = E // NSUB
NCHUNK = EPT // EC
EPS = 1e-16


def _ceil_div(a, b):
    return (a + b - 1) // b


@functools.lru_cache(maxsize=None)
def _conv_kernel(nsrc, ndst):
    half = ndst // 2

    def body(hs_hbm, ss_hbm, sd_hbm, src_hbm, dst_hbm, num_out, den_out,
             sidx, didx, lidx, ssv, sdv, exv, rows, msg, stg,
             ss_sh, sd_sh, num_sh, den_sh, sem, sem2):
        core = lax.axis_index("c")
        sub = lax.axis_index("s")

        nss = nsrc // SC_CH
        for k in range(_ceil_div(nss, NSUB)):
            c = sub + NSUB * k

            @pl.when(c < nss)
            def _():
                off = c * SC_CH
                pltpu.sync_copy(ss_hbm.at[pl.ds(off, SC_CH)],
                                stg.at[pl.ds(0, SC_CH)])
                pltpu.sync_copy(stg.at[pl.ds(0, SC_CH)],
                                ss_sh.at[pl.ds(off, SC_CH)])

        nsd = ndst // SC_CH
        for k in range(_ceil_div(nsd, NSUB)):
            c = sub + NSUB * k

            @pl.when(c < nsd)
            def _():
                off = c * SC_CH
                pltpu.sync_copy(sd_hbm.at[pl.ds(off, SC_CH)],
                                stg.at[pl.ds(0, SC_CH)])
                pltpu.sync_copy(stg.at[pl.ds(0, SC_CH)],
                                sd_sh.at[pl.ds(off, SC_CH)])

        zv = jnp.zeros((L,), jnp.float32)

        def zs(i, _):
            stg[pl.ds(i * L, L)] = zv
            return 0
        lax.fori_loop(0, stg.shape[0] // L, zs, 0)

        def zm(i, _):
            msg[i, pl.ds(0, L)] = zv
            msg[i, pl.ds(L, L)] = zv
            return 0
        lax.fori_loop(0, WR_CH, zm, 0)

        ndc = half // SC_CH
        for k in range(_ceil_div(ndc, NSUB)):
            c = sub + NSUB * k

            @pl.when(c < ndc)
            def _():
                pltpu.sync_copy(stg.at[pl.ds(0, SC_CH)],
                                den_sh.at[pl.ds(c * SC_CH, SC_CH)])

        nwc = half // WR_CH
        for k in range(_ceil_div(nwc, NSUB)):
            c = sub + NSUB * k

            @pl.when(c < nwc)
            def _():
                pltpu.sync_copy(msg, num_sh.at[pl.ds(c * WR_CH, WR_CH)])

        plsc.subcore_barrier()

        base = core * half

        def chunk(k, _):
            off = (k * NSUB + sub) * EC
            pltpu.sync_copy(dst_hbm.at[pl.ds(off, EC)], didx)
            lo = didx[pl.ds(0, L)][0]
            hi = didx[pl.ds(EC - L, L)][L - 1]

            @pl.when((hi >= base) & (lo < base + half))
            def _():
                pltpu.sync_copy(src_hbm.at[pl.ds(off, EC)], sidx)
                c1 = pltpu.async_copy(ss_sh.at[sidx], ssv, sem)
                c2 = pltpu.async_copy(sd_sh.at[didx], sdv, sem)
                c3 = pltpu.async_copy(hs_hbm.at[sidx], rows, sem2)
                c1.wait()
                c2.wait()
                c3.wait()

                def grp(g, _):
                    sv = ssv[pl.ds(g * L, L)]
                    dv = sdv[pl.ds(g * L, L)]
                    s = sv + dv
                    e = jnp.where(s >= 0, s, 0.2 * s)
                    ex = jnp.exp(e)
                    exv[pl.ds(g * L, L)] = ex
                    dl = didx[pl.ds(g * L, L)] - base
                    okm = (dl >= 0) & (dl < half)
                    lidx[0, pl.ds(g * L, L)] = jnp.where(okm, dl, half)
                    for j in range(L):
                        sj = ex[j]
                        ei = g * L + j
                        msg[ei, pl.ds(0, L)] = sj * rows[ei, pl.ds(0, L)]
                        msg[ei, pl.ds(L, L)] = sj * rows[ei, pl.ds(L, L)]
                    return 0

                lax.fori_loop(0, EC // L, grp, 0)
                pltpu.sync_copy(msg.at[pl.ds(0, EC)], num_sh.at[lidx.at[0]],
                                add=True)
                pltpu.sync_copy(exv, den_sh.at[lidx.at[0]], add=True)

            return 0

        lax.fori_loop(0, NCHUNK, chunk, 0)
        plsc.subcore_barrier()

        for k in range(_ceil_div(nwc, NSUB)):
            c = sub + NSUB * k

            @pl.when(c < nwc)
            def _():
                r0 = c * WR_CH
                pltpu.sync_copy(num_sh.at[pl.ds(r0, WR_CH)], msg)
                pltpu.sync_copy(msg, num_out.at[pl.ds(base + r0, WR_CH)])

        for k in range(_ceil_div(ndc, NSUB)):
            c = sub + NSUB * k

            @pl.when(c < ndc)
            def _():
                r0 = c * SC_CH
                pltpu.sync_copy(den_sh.at[pl.ds(r0, SC_CH)],
                                stg.at[pl.ds(0, SC_CH)])
                pltpu.sync_copy(stg.at[pl.ds(0, SC_CH)],
                                den_out.at[pl.ds(base + r0, SC_CH)])

    mesh = plsc.VectorSubcoreMesh(core_axis_name="c", subcore_axis_name="s",
                                  num_cores=NCORE, num_subcores=NSUB)
    return pl.kernel(
        body,
        out_type=[jax.ShapeDtypeStruct((ndst, HID), jnp.float32),
                  jax.ShapeDtypeStruct((ndst,), jnp.float32)],
        mesh=mesh,
        compiler_params=pltpu.CompilerParams(use_tc_tiling_on_sc=False),
        scratch_types=[
            pltpu.VMEM((EC,), jnp.int32),
            pltpu.VMEM((EC,), jnp.int32),
            pltpu.VMEM((1, EC), jnp.int32),
            pltpu.VMEM((EC,), jnp.float32),
            pltpu.VMEM((EC,), jnp.float32),
            pltpu.VMEM((EC,), jnp.float32),
            pltpu.VMEM((EC, HID), jnp.float32),
            pltpu.VMEM((WR_CH, HID), jnp.float32),
            pltpu.VMEM((1008,), jnp.float32),
            pltpu.MemorySpace.VMEM_SHARED((nsrc,), jnp.float32),
            pltpu.MemorySpace.VMEM_SHARED((ndst,), jnp.float32),
            pltpu.MemorySpace.VMEM_SHARED((ndst // 2 + 8, HID), jnp.float32),
            pltpu.MemorySpace.VMEM_SHARED((ndst // 2 + 8,), jnp.float32),
            pltpu.SemaphoreType.DMA,
            pltpu.SemaphoreType.DMA,
        ],
    )


_B = 1000


@functools.lru_cache(maxsize=None)
def _prep_kernel(n, parts):
    np_ = len(parts)

    def body(x_ref, w_ref, a_ref, *outs):
        x = x_ref[...]
        y = jnp.dot(x, w_ref[...], preferred_element_type=jnp.float32)
        oi = 0
        for i, kind in enumerate(parts):
            h0 = y[:, i * 2 * HID:i * 2 * HID + HID]
            h1 = y[:, i * 2 * HID + HID:(i + 1) * 2 * HID]
            s0 = jnp.sum(h0 * a_ref[2 * i][None, :], axis=1)
            s1 = jnp.sum(h1 * a_ref[2 * i + 1][None, :], axis=1)
            if kind == "p":
                outs[oi][...] = h0
                outs[oi + 1][...] = h1
                oi += 2
            outs[oi][...] = s0[:, None]
            outs[oi + 1][...] = s1[:, None]
            oi += 2

    out_shape = []
    out_specs = []
    for kind in parts:
        if kind == "p":
            for _ in range(H):
                out_shape.append(jax.ShapeDtypeStruct((n, HID), jnp.float32))
                out_specs.append(pl.BlockSpec((_B, HID), lambda i: (i, 0)))
        for _ in range(H):
            out_shape.append(jax.ShapeDtypeStruct((n, 1), jnp.float32))
            out_specs.append(pl.BlockSpec((_B, 1), lambda i: (i, 0)))

    return pl.pallas_call(
        body,
        grid=(n // _B,),
        in_specs=[
            pl.BlockSpec((_B, HID), lambda i: (i, 0)),
            pl.BlockSpec((HID, 2 * HID * np_), lambda i: (0, 0)),
            pl.BlockSpec((2 * np_, HID), lambda i: (0, 0)),
        ],
        out_specs=out_specs,
        out_shape=out_shape,
    )


@functools.lru_cache(maxsize=None)
def _combine_kernel(n, nconv, lin):
    def body(*refs):
        i = 0
        nums = []
        dens = []
        for _ in range(nconv * H):
            nums.append(refs[i])
            i += 1
        for _ in range(nconv * H):
            dens.append(refs[i])
            i += 1
        bsum = refs[i][...]
        i += 1
        if lin:
            lw = refs[i][...]
            lb = refs[i + 1][...]
            i += 2
        out = refs[i]
        pre = jnp.broadcast_to(bsum, (_B, HID)).astype(jnp.float32)
        for c in range(nconv):
            m0 = nums[2 * c][...] / (dens[2 * c][...] + EPS)
            m1 = nums[2 * c + 1][...] / (dens[2 * c + 1][...] + EPS)
            pre = pre + 0.5 * (m0 + m1)
        if lin:
            pre = jnp.dot(pre, lw, preferred_element_type=jnp.float32) + lb
        out[...] = jnp.maximum(pre, 0.0)

    in_specs = (
        [pl.BlockSpec((_B, HID), lambda i: (i, 0))] * (nconv * H)
        + [pl.BlockSpec((_B, 1), lambda i: (i, 0))] * (nconv * H)
        + [pl.BlockSpec((1, HID), lambda i: (0, 0))]
    )
    if lin:
        in_specs += [pl.BlockSpec((HID, HID), lambda i: (0, 0)),
                     pl.BlockSpec((1, HID), lambda i: (0, 0))]

    return pl.pallas_call(
        body,
        grid=(n // _B,),
        in_specs=in_specs,
        out_specs=pl.BlockSpec((_B, HID), lambda i: (i, 0)),
        out_shape=jax.ShapeDtypeStruct((n, HID), jnp.float32),
    )


def _gat_pair(hs, ss, sd, src, dst, nsrc, ndst):
    f = _conv_kernel(nsrc, ndst)
    nums = []
    dens = []
    for h in range(H):
        num, den = f(hs[h], ss[h].reshape(nsrc), sd[h].reshape(ndst),
                     src, dst)
        nums.append(num)
        dens.append(den.reshape(ndst, 1))
    return nums, dens


def _layer(xu, xb, xm, w, pfx, e_ub, e_um, e_bu, e_mu, lin=None):
    def cat(names_w, names_a):
        return (jnp.concatenate([w[pfx + n + "_W" + k] for n, k in names_w],
                                axis=1),
                jnp.concatenate([w[pfx + n + "_a" + k] for n, k in names_a],
                                axis=0))

    wu, au = cat([("ub", "s"), ("um", "s"), ("bu", "d"), ("mu", "d")],
                 [("ub", "s"), ("um", "s"), ("bu", "d"), ("mu", "d")])
    ou_ = _prep_kernel(NU, "ppss")(xu, wu, au)
    hs_ub, ss_ub = ou_[0:2], ou_[2:4]
    hs_um, ss_um = ou_[4:6], ou_[6:8]
    sd_bu, sd_mu = ou_[8:10], ou_[10:12]
    wb, ab = cat([("bu", "s"), ("ub", "d")], [("bu", "s"), ("ub", "d")])
    ob_ = _prep_kernel(NB, "ps")(xb, wb, ab)
    hs_bu, ss_bu, sd_ub = ob_[0:2], ob_[2:4], ob_[4:6]
    wm, am = cat([("mu", "s"), ("um", "d")], [("mu", "s"), ("um", "d")])
    om_ = _prep_kernel(NM, "ps")(xm, wm, am)
    hs_mu, ss_mu, sd_um = om_[0:2], om_[2:4], om_[4:6]

    nb_n, nb_d = _gat_pair(hs_ub, ss_ub, sd_ub, e_ub[0], e_ub[1], NU, NB)
    nm_n, nm_d = _gat_pair(hs_um, ss_um, sd_um, e_um[0], e_um[1], NU, NM)
    ub_n, ub_d = _gat_pair(hs_bu, ss_bu, sd_bu, e_bu[0], e_bu[1], NB, NU)
    um_n, um_d = _gat_pair(hs_mu, ss_mu, sd_mu, e_mu[0], e_mu[1], NM, NU)

    b_b = w[pfx + "ub_b"].reshape(1, HID)
    b_m = w[pfx + "um_b"].reshape(1, HID)
    b_u = (w[pfx + "bu_b"] + w[pfx + "mu_b"]).reshape(1, HID)
    if lin is None:
        cb = _combine_kernel(NB, 1, False)
        cu = _combine_kernel(NU, 2, False)
        ob = cb(*nb_n, *nb_d, b_b)
        om = cb(*nm_n, *nm_d, b_m)
        ou = cu(*ub_n, *um_n, *ub_d, *um_d, b_u)
    else:
        cb = _combine_kernel(NB, 1, True)
        cu = _combine_kernel(NU, 2, True)
        ob = cb(*nb_n, *nb_d, b_b, lin["b_W"], lin["b_b"].reshape(1, HID))
        om = cb(*nm_n, *nm_d, b_m, lin["m_W"], lin["m_b"].reshape(1, HID))
        ou = cu(*ub_n, *um_n, *ub_d, *um_d, b_u,
                lin["u_W"], lin["u_b"].reshape(1, HID))
    return ou, ob, om


def kernel(edge_index_rates_book, edge_index_rates_movie, user_emb, book_emb,
           movie_emb,
           l1_ub_Ws, l1_ub_Wd, l1_ub_as, l1_ub_ad, l1_ub_b,
           l1_bu_Ws, l1_bu_Wd, l1_bu_as, l1_bu_ad, l1_bu_b,
           l1_um_Ws, l1_um_Wd, l1_um_as, l1_um_ad, l1_um_b,
           l1_mu_Ws, l1_mu_Wd, l1_mu_as, l1_mu_ad, l1_mu_b,
           l2_ub_Ws, l2_ub_Wd, l2_ub_as, l2_ub_ad, l2_ub_b,
           l2_bu_Ws, l2_bu_Wd, l2_bu_as, l2_bu_ad, l2_bu_b,
           l2_um_Ws, l2_um_Wd, l2_um_as, l2_um_ad, l2_um_b,
           l2_mu_Ws, l2_mu_Wd, l2_mu_as, l2_mu_ad, l2_mu_b,
           lin_u_W, lin_u_b, lin_b_W, lin_b_b, lin_m_W, lin_m_b):
    w = dict(locals())
    wmap = {}
    for key, val in w.items():
        wmap[key] = val
    conv_w = {k: v for k, v in wmap.items()}

    ei_ub = edge_index_rates_book
    ei_um = edge_index_rates_movie

    def bydst(src, dst):
        perm = jnp.argsort(dst)
        return src[perm], dst[perm]

    e_ub = bydst(ei_ub[0], ei_ub[1])
    e_um = bydst(ei_um[0], ei_um[1])
    e_bu = bydst(ei_ub[1], ei_ub[0])
    e_mu = bydst(ei_um[1], ei_um[0])

    u1, b1, m1 = _layer(user_emb, book_emb, movie_emb, conv_w, "l1_",
                        e_ub, e_um, e_bu, e_mu)
    lin = {"u_W": lin_u_W, "u_b": lin_u_b, "b_W": lin_b_W, "b_b": lin_b_b,
           "m_W": lin_m_W, "m_b": lin_m_b}
    ou, ob, om = _layer(u1, b1, m1, conv_w, "l2_", e_ub, e_um, e_bu, e_mu,
                        lin=lin)
    return jnp.concatenate([ou, ob, om], axis=0)


# speedup vs baseline: 46.8383x; 1.2920x over previous
import functools
import jax
import jax.numpy as jnp
from jax import lax
from jax.experimental import pallas as pl
from jax.experimental.pallas import tpu as pltpu
from jax.experimental.pallas import tpu_sc as plsc

H = 2
HID = 32
NU = 100000
NB = 50000
NM = 50000
E = 800000

NCORE, NSUB, L = 2, 16, 16

EC = 160
SC_CH = 1000
WR_CH = 250
NCH_G = E // EC
NCHUNK = (NCH_G + NSUB - 1) // NSUB
EPS = 1e-16


def _ceil_div(a, b):
    return (a + b - 1) // b


@functools.lru_cache(maxsize=None)
def _conv_kernel(nsrc, ndst):
    half = ndst // 2

    def body(hs_hbm, ss_hbm, sd_hbm, src_hbm, dst_hbm, num_out, den_out,
             sidx, didx, lidx, ssv, sdv, exv, rows, msg, stg,
             ss_sh, sd_sh, num_sh, den_sh, sem, sem2):
        core = lax.axis_index("c")
        sub = lax.axis_index("s")

        nss = nsrc // SC_CH
        for k in range(_ceil_div(nss, NSUB)):
            c = sub + NSUB * k

            @pl.when(c < nss)
            def _():
                off = c * SC_CH
                pltpu.sync_copy(ss_hbm.at[pl.ds(off, SC_CH)],
                                stg.at[pl.ds(0, SC_CH)])
                pltpu.sync_copy(stg.at[pl.ds(0, SC_CH)],
                                ss_sh.at[pl.ds(off, SC_CH)])

        nsd = ndst // SC_CH
        for k in range(_ceil_div(nsd, NSUB)):
            c = sub + NSUB * k

            @pl.when(c < nsd)
            def _():
                off = c * SC_CH
                pltpu.sync_copy(sd_hbm.at[pl.ds(off, SC_CH)],
                                stg.at[pl.ds(0, SC_CH)])
                pltpu.sync_copy(stg.at[pl.ds(0, SC_CH)],
                                sd_sh.at[pl.ds(off, SC_CH)])

        zv = jnp.zeros((L,), jnp.float32)

        def zs(i, _):
            stg[pl.ds(i * L, L)] = zv
            return 0
        lax.fori_loop(0, stg.shape[0] // L, zs, 0)

        def zm(i, _):
            msg[i, pl.ds(0, L)] = zv
            msg[i, pl.ds(L, L)] = zv
            return 0
        lax.fori_loop(0, WR_CH, zm, 0)

        ndc = half // SC_CH
        for k in range(_ceil_div(ndc, NSUB)):
            c = sub + NSUB * k

            @pl.when(c < ndc)
            def _():
                pltpu.sync_copy(stg.at[pl.ds(0, SC_CH)],
                                den_sh.at[pl.ds(c * SC_CH, SC_CH)])

        nwc = half // WR_CH
        for k in range(_ceil_div(nwc, NSUB)):
            c = sub + NSUB * k

            @pl.when(c < nwc)
            def _():
                pltpu.sync_copy(msg, num_sh.at[pl.ds(c * WR_CH, WR_CH)])

        plsc.subcore_barrier()

        base = core * half

        def chunk(k, _):
            c = k * NSUB + sub

            @pl.when(c < NCH_G)
            def _():
                _chunk_body(c)
            return 0

        def _chunk_body(c):
            off = c * EC
            pltpu.sync_copy(dst_hbm.at[pl.ds(off, EC)], didx)
            lo = didx[pl.ds(0, L)][0]
            hi = didx[pl.ds(EC - L, L)][L - 1]

            @pl.when((hi >= base) & (lo < base + half))
            def _():
                pltpu.sync_copy(src_hbm.at[pl.ds(off, EC)], sidx)
                c1 = pltpu.async_copy(ss_sh.at[sidx], ssv, sem)
                c2 = pltpu.async_copy(sd_sh.at[didx], sdv, sem)
                c3 = pltpu.async_copy(hs_hbm.at[sidx], rows, sem2)
                c1.wait()
                c2.wait()
                c3.wait()

                def grp(g, _):
                    sv = ssv[pl.ds(g * L, L)]
                    dv = sdv[pl.ds(g * L, L)]
                    s = sv + dv
                    e = jnp.where(s >= 0, s, 0.2 * s)
                    ex = jnp.exp(e)
                    exv[pl.ds(g * L, L)] = ex
                    dl = didx[pl.ds(g * L, L)] - base
                    okm = (dl >= 0) & (dl < half)
                    lidx[0, pl.ds(g * L, L)] = jnp.where(okm, dl, half)
                    for j in range(L):
                        sj = ex[j]
                        ei = g * L + j
                        msg[ei, pl.ds(0, L)] = sj * rows[ei, pl.ds(0, L)]
                        msg[ei, pl.ds(L, L)] = sj * rows[ei, pl.ds(L, L)]
                    return 0

                lax.fori_loop(0, EC // L, grp, 0)
                pltpu.sync_copy(msg.at[pl.ds(0, EC)], num_sh.at[lidx.at[0]],
                                add=True)
                pltpu.sync_copy(exv, den_sh.at[lidx.at[0]], add=True)

            return 0

        lax.fori_loop(0, NCHUNK, chunk, 0)
        plsc.subcore_barrier()

        for k in range(_ceil_div(nwc, NSUB)):
            c = sub + NSUB * k

            @pl.when(c < nwc)
            def _():
                r0 = c * WR_CH
                pltpu.sync_copy(num_sh.at[pl.ds(r0, WR_CH)], msg)
                pltpu.sync_copy(msg, num_out.at[pl.ds(base + r0, WR_CH)])

        for k in range(_ceil_div(ndc, NSUB)):
            c = sub + NSUB * k

            @pl.when(c < ndc)
            def _():
                r0 = c * SC_CH
                pltpu.sync_copy(den_sh.at[pl.ds(r0, SC_CH)],
                                stg.at[pl.ds(0, SC_CH)])
                pltpu.sync_copy(stg.at[pl.ds(0, SC_CH)],
                                den_out.at[pl.ds(base + r0, SC_CH)])

    mesh = plsc.VectorSubcoreMesh(core_axis_name="c", subcore_axis_name="s",
                                  num_cores=NCORE, num_subcores=NSUB)
    return pl.kernel(
        body,
        out_type=[jax.ShapeDtypeStruct((ndst, HID), jnp.float32),
                  jax.ShapeDtypeStruct((ndst,), jnp.float32)],
        mesh=mesh,
        compiler_params=pltpu.CompilerParams(use_tc_tiling_on_sc=False),
        scratch_types=[
            pltpu.VMEM((EC,), jnp.int32),
            pltpu.VMEM((EC,), jnp.int32),
            pltpu.VMEM((1, EC), jnp.int32),
            pltpu.VMEM((EC,), jnp.float32),
            pltpu.VMEM((EC,), jnp.float32),
            pltpu.VMEM((EC,), jnp.float32),
            pltpu.VMEM((EC, HID), jnp.float32),
            pltpu.VMEM((WR_CH, HID), jnp.float32),
            pltpu.VMEM((1008,), jnp.float32),
            pltpu.MemorySpace.VMEM_SHARED((nsrc,), jnp.float32),
            pltpu.MemorySpace.VMEM_SHARED((ndst,), jnp.float32),
            pltpu.MemorySpace.VMEM_SHARED((ndst // 2 + 8, HID), jnp.float32),
            pltpu.MemorySpace.VMEM_SHARED((ndst // 2 + 8,), jnp.float32),
            pltpu.SemaphoreType.DMA,
            pltpu.SemaphoreType.DMA,
        ],
    )


_B = 1000


@functools.lru_cache(maxsize=None)
def _prep_kernel(n, parts):
    np_ = len(parts)

    def body(x_ref, w_ref, a_ref, *outs):
        x = x_ref[...]
        y = jnp.dot(x, w_ref[...], preferred_element_type=jnp.float32)
        oi = 0
        for i, kind in enumerate(parts):
            h0 = y[:, i * 2 * HID:i * 2 * HID + HID]
            h1 = y[:, i * 2 * HID + HID:(i + 1) * 2 * HID]
            s0 = jnp.sum(h0 * a_ref[2 * i][None, :], axis=1)
            s1 = jnp.sum(h1 * a_ref[2 * i + 1][None, :], axis=1)
            if kind == "p":
                outs[oi][...] = h0
                outs[oi + 1][...] = h1
                oi += 2
            outs[oi][...] = s0[:, None]
            outs[oi + 1][...] = s1[:, None]
            oi += 2

    out_shape = []
    out_specs = []
    for kind in parts:
        if kind == "p":
            for _ in range(H):
                out_shape.append(jax.ShapeDtypeStruct((n, HID), jnp.float32))
                out_specs.append(pl.BlockSpec((_B, HID), lambda i: (i, 0)))
        for _ in range(H):
            out_shape.append(jax.ShapeDtypeStruct((n, 1), jnp.float32))
            out_specs.append(pl.BlockSpec((_B, 1), lambda i: (i, 0)))

    return pl.pallas_call(
        body,
        grid=(n // _B,),
        in_specs=[
            pl.BlockSpec((_B, HID), lambda i: (i, 0)),
            pl.BlockSpec((HID, 2 * HID * np_), lambda i: (0, 0)),
            pl.BlockSpec((2 * np_, HID), lambda i: (0, 0)),
        ],
        out_specs=out_specs,
        out_shape=out_shape,
    )


@functools.lru_cache(maxsize=None)
def _combine_kernel(n, nconv, lin):
    def body(*refs):
        i = 0
        nums = []
        dens = []
        for _ in range(nconv * H):
            nums.append(refs[i])
            i += 1
        for _ in range(nconv * H):
            dens.append(refs[i])
            i += 1
        bsum = refs[i][...]
        i += 1
        if lin:
            lw = refs[i][...]
            lb = refs[i + 1][...]
            i += 2
        out = refs[i]
        pre = jnp.broadcast_to(bsum, (_B, HID)).astype(jnp.float32)
        for c in range(nconv):
            m0 = nums[2 * c][...] / (dens[2 * c][...] + EPS)
            m1 = nums[2 * c + 1][...] / (dens[2 * c + 1][...] + EPS)
            pre = pre + 0.5 * (m0 + m1)
        if lin:
            pre = jnp.dot(pre, lw, preferred_element_type=jnp.float32) + lb
        out[...] = jnp.maximum(pre, 0.0)

    in_specs = (
        [pl.BlockSpec((_B, HID), lambda i: (i, 0))] * (nconv * H)
        + [pl.BlockSpec((_B, 1), lambda i: (i, 0))] * (nconv * H)
        + [pl.BlockSpec((1, HID), lambda i: (0, 0))]
    )
    if lin:
        in_specs += [pl.BlockSpec((HID, HID), lambda i: (0, 0)),
                     pl.BlockSpec((1, HID), lambda i: (0, 0))]

    return pl.pallas_call(
        body,
        grid=(n // _B,),
        in_specs=in_specs,
        out_specs=pl.BlockSpec((_B, HID), lambda i: (i, 0)),
        out_shape=jax.ShapeDtypeStruct((n, HID), jnp.float32),
    )


def _gat_pair(hs, ss, sd, src, dst, nsrc, ndst):
    f = _conv_kernel(nsrc, ndst)
    nums = []
    dens = []
    for h in range(H):
        num, den = f(hs[h], ss[h].reshape(nsrc), sd[h].reshape(ndst),
                     src, dst)
        nums.append(num)
        dens.append(den.reshape(ndst, 1))
    return nums, dens


def _layer(xu, xb, xm, w, pfx, e_ub, e_um, e_bu, e_mu, lin=None):
    def cat(names_w, names_a):
        return (jnp.concatenate([w[pfx + n + "_W" + k] for n, k in names_w],
                                axis=1),
                jnp.concatenate([w[pfx + n + "_a" + k] for n, k in names_a],
                                axis=0))

    wu, au = cat([("ub", "s"), ("um", "s"), ("bu", "d"), ("mu", "d")],
                 [("ub", "s"), ("um", "s"), ("bu", "d"), ("mu", "d")])
    ou_ = _prep_kernel(NU, "ppss")(xu, wu, au)
    hs_ub, ss_ub = ou_[0:2], ou_[2:4]
    hs_um, ss_um = ou_[4:6], ou_[6:8]
    sd_bu, sd_mu = ou_[8:10], ou_[10:12]
    wb, ab = cat([("bu", "s"), ("ub", "d")], [("bu", "s"), ("ub", "d")])
    ob_ = _prep_kernel(NB, "ps")(xb, wb, ab)
    hs_bu, ss_bu, sd_ub = ob_[0:2], ob_[2:4], ob_[4:6]
    wm, am = cat([("mu", "s"), ("um", "d")], [("mu", "s"), ("um", "d")])
    om_ = _prep_kernel(NM, "ps")(xm, wm, am)
    hs_mu, ss_mu, sd_um = om_[0:2], om_[2:4], om_[4:6]

    nb_n, nb_d = _gat_pair(hs_ub, ss_ub, sd_ub, e_ub[0], e_ub[1], NU, NB)
    nm_n, nm_d = _gat_pair(hs_um, ss_um, sd_um, e_um[0], e_um[1], NU, NM)
    ub_n, ub_d = _gat_pair(hs_bu, ss_bu, sd_bu, e_bu[0], e_bu[1], NB, NU)
    um_n, um_d = _gat_pair(hs_mu, ss_mu, sd_mu, e_mu[0], e_mu[1], NM, NU)

    b_b = w[pfx + "ub_b"].reshape(1, HID)
    b_m = w[pfx + "um_b"].reshape(1, HID)
    b_u = (w[pfx + "bu_b"] + w[pfx + "mu_b"]).reshape(1, HID)
    if lin is None:
        cb = _combine_kernel(NB, 1, False)
        cu = _combine_kernel(NU, 2, False)
        ob = cb(*nb_n, *nb_d, b_b)
        om = cb(*nm_n, *nm_d, b_m)
        ou = cu(*ub_n, *um_n, *ub_d, *um_d, b_u)
    else:
        cb = _combine_kernel(NB, 1, True)
        cu = _combine_kernel(NU, 2, True)
        ob = cb(*nb_n, *nb_d, b_b, lin["b_W"], lin["b_b"].reshape(1, HID))
        om = cb(*nm_n, *nm_d, b_m, lin["m_W"], lin["m_b"].reshape(1, HID))
        ou = cu(*ub_n, *um_n, *ub_d, *um_d, b_u,
                lin["u_W"], lin["u_b"].reshape(1, HID))
    return ou, ob, om


def kernel(edge_index_rates_book, edge_index_rates_movie, user_emb, book_emb,
           movie_emb,
           l1_ub_Ws, l1_ub_Wd, l1_ub_as, l1_ub_ad, l1_ub_b,
           l1_bu_Ws, l1_bu_Wd, l1_bu_as, l1_bu_ad, l1_bu_b,
           l1_um_Ws, l1_um_Wd, l1_um_as, l1_um_ad, l1_um_b,
           l1_mu_Ws, l1_mu_Wd, l1_mu_as, l1_mu_ad, l1_mu_b,
           l2_ub_Ws, l2_ub_Wd, l2_ub_as, l2_ub_ad, l2_ub_b,
           l2_bu_Ws, l2_bu_Wd, l2_bu_as, l2_bu_ad, l2_bu_b,
           l2_um_Ws, l2_um_Wd, l2_um_as, l2_um_ad, l2_um_b,
           l2_mu_Ws, l2_mu_Wd, l2_mu_as, l2_mu_ad, l2_mu_b,
           lin_u_W, lin_u_b, lin_b_W, lin_b_b, lin_m_W, lin_m_b):
    w = dict(locals())
    wmap = {}
    for key, val in w.items():
        wmap[key] = val
    conv_w = {k: v for k, v in wmap.items()}

    ei_ub = edge_index_rates_book
    ei_um = edge_index_rates_movie

    def bydst(src, dst):
        perm = jnp.argsort(dst)
        return src[perm], dst[perm]

    e_ub = bydst(ei_ub[0], ei_ub[1])
    e_um = bydst(ei_um[0], ei_um[1])
    e_bu = bydst(ei_ub[1], ei_ub[0])
    e_mu = bydst(ei_um[1], ei_um[0])

    u1, b1, m1 = _layer(user_emb, book_emb, movie_emb, conv_w, "l1_",
                        e_ub, e_um, e_bu, e_mu)
    lin = {"u_W": lin_u_W, "u_b": lin_u_b, "b_W": lin_b_W, "b_b": lin_b_b,
           "m_W": lin_m_W, "m_b": lin_m_b}
    ou, ob, om = _layer(u1, b1, m1, conv_w, "l2_", e_ub, e_um, e_bu, e_mu,
                        lin=lin)
    return jnp.concatenate([ou, ob, om], axis=0)
